# baseline (speedup 1.0000x reference)
import functools

import jax
import jax.numpy as jnp
from jax import lax
from jax.experimental import pallas as pl
from jax.experimental.pallas import tpu as pltpu
from jax.experimental.pallas import tpu_sc as plsc

NW = 32
CK = 2000
G = 32
LSZ = CK + 2 * G


def _sc_agg_build(E, N, C):
  R = -(-N // NW)
  assert (R * C) % 16 == 0 and E % CK == 0
  mesh = plsc.VectorSubcoreMesh(core_axis_name="c", subcore_axis_name="s")

  @functools.partial(
      pl.kernel,
      mesh=mesh,
      compiler_params=pltpu.CompilerParams(needs_layout_passes=False, use_tc_tiling_on_sc=False),
      out_type=jax.ShapeDtypeStruct((NW * R * C,), jnp.float32),
      scratch_types=[
          pltpu.VMEM((CK,), jnp.int32),
          pltpu.VMEM((CK,), jnp.int32),
          pltpu.VMEM((CK,), jnp.float32),
          pltpu.VMEM((LSZ,), jnp.int32),
          pltpu.VMEM((LSZ,), jnp.int32),
          pltpu.VMEM((LSZ,), jnp.float32),
          pltpu.VMEM((G, C), jnp.float32),
          pltpu.VMEM((R * C,), jnp.float32),
          pltpu.SemaphoreType.DMA,
      ],
  )
  def agg_kernel(src_hbm, dst_hbm, w_hbm, sup_hbm, out_hbm,
                 dstb, srcb, wb, ld, ls, lw, rows, acc, sem):
    wid = lax.axis_index("s") * 2 + lax.axis_index("c")
    lo = wid * R
    z16i = jnp.zeros((16,), jnp.int32)
    z16f = jnp.zeros((16,), jnp.float32)
    iota16 = lax.iota(jnp.int32, 16)

    def zero(i, carry):
      acc[pl.ds(i * 16, 16)] = z16f
      return carry
    lax.fori_loop(0, (R * C) // 16, zero, 0)

    def chunk(c, carry):
      pltpu.sync_copy(dst_hbm.at[pl.ds(c * CK, CK)], dstb)
      pltpu.sync_copy(src_hbm.at[pl.ds(c * CK, CK)], srcb)
      pltpu.sync_copy(w_hbm.at[pl.ds(c * CK, CK)], wb)

      def compact(i, pos):
        d = dstb[pl.ds(i * 16, 16)]
        loc = d - lo
        m = (loc >= 0) & (loc < R)
        mi = jnp.where(m, z16i + 1, z16i)
        posv = plsc.cumsum(mi) - 1 + pos
        plsc.store_scatter(ld, [posv], loc, mask=m)
        plsc.store_scatter(ls, [posv], srcb[pl.ds(i * 16, 16)], mask=m)
        plsc.store_scatter(lw, [posv], wb[pl.ds(i * 16, 16)], mask=m)
        return pos + jnp.sum(mi)
      pos = lax.fori_loop(0, CK // 16, compact, jnp.int32(0))

      for k in range(G // 16):
        ld[pl.ds(pos + k * 16, 16)] = z16i
        ls[pl.ds(pos + k * 16, 16)] = z16i
        lw[pl.ds(pos + k * 16, 16)] = z16f

      nb = (pos + (G - 1)) // G

      def block(b, carry):
        pltpu.async_copy(sup_hbm.at[ls.at[pl.ds(b * G, G)]], rows, sem).wait()

        def edge(e, carry2):
          espl = z16i + (b * G + e)
          wspl = plsc.load_gather(lw, [espl])
          dspl = plsc.load_gather(ld, [espl])
          dbase = dspl * C
          el = z16i + e
          for g in range(C // 16):
            col = iota16 + g * 16
            v = plsc.load_gather(rows, [el, col])
            plsc.addupdate_scatter(acc, [dbase + col], v * wspl)
          return carry2
        lax.fori_loop(0, G, edge, 0)
        return carry
      lax.fori_loop(0, nb, block, 0)
      return carry
    lax.fori_loop(0, E // CK, chunk, 0)

    pltpu.sync_copy(acc, out_hbm.at[pl.ds(wid * (R * C), R * C)])

  return agg_kernel, R


def _sc_agg(src, dst, w, sup):
  E = src.shape[0]
  N, C = sup.shape
  fn, R = _sc_agg_build(E, N, C)
  out = fn(src, dst, w, sup)
  return out.reshape(NW * R, C)[:N]


def _tc1(x, W1, selr):
  N, F = x.shape
  H = W1.shape[1]
  BN = N // 10

  def body(x_ref, w_ref, s_ref, sup_ref, fs_ref):
    fs = jax.nn.sigmoid(s_ref[...])
    fs_ref[...] = fs
    sup_ref[...] = jnp.dot(x_ref[...] * fs, w_ref[...],
                           preferred_element_type=jnp.float32)

  return pl.pallas_call(
      body,
      grid=(N // BN,),
      in_specs=[
          pl.BlockSpec((BN, F), lambda i: (i, 0)),
          pl.BlockSpec((F, H), lambda i: (0, 0)),
          pl.BlockSpec((1, F), lambda i: (0, 0)),
      ],
      out_specs=[
          pl.BlockSpec((BN, H), lambda i: (i, 0)),
          pl.BlockSpec((1, F), lambda i: (0, 0)),
      ],
      out_shape=[
          jax.ShapeDtypeStruct((N, H), jnp.float32),
          jax.ShapeDtypeStruct((1, F), jnp.float32),
      ],
  )(x, W1, selr)


def _tc2(agg, b1, W2):
  N, H = agg.shape
  K = W2.shape[1]
  BN = N // 10

  def body(a_ref, b_ref, w_ref, e1_ref, s2_ref):
    e1 = a_ref[...] + b_ref[...]
    e1_ref[...] = e1
    s2_ref[...] = jnp.dot(jnp.maximum(e1, 0.0), w_ref[...],
                          preferred_element_type=jnp.float32)

  return pl.pallas_call(
      body,
      grid=(N // BN,),
      in_specs=[
          pl.BlockSpec((BN, H), lambda i: (i, 0)),
          pl.BlockSpec((1, H), lambda i: (0, 0)),
          pl.BlockSpec((H, K), lambda i: (0, 0)),
      ],
      out_specs=[
          pl.BlockSpec((BN, H), lambda i: (i, 0)),
          pl.BlockSpec((BN, K), lambda i: (i, 0)),
      ],
      out_shape=[
          jax.ShapeDtypeStruct((N, H), jnp.float32),
          jax.ShapeDtypeStruct((N, K), jnp.float32),
      ],
  )(agg, b1, W2)


def _tc3(agg2, b2):
  N, K = agg2.shape
  BN = N // 10

  def body(a_ref, b_ref, e2_ref, lp_ref):
    e2 = a_ref[...] + b_ref[...]
    e2_ref[...] = e2
    m = jnp.max(e2, axis=1, keepdims=True)
    lse = jnp.log(jnp.sum(jnp.exp(e2 - m), axis=1, keepdims=True)) + m
    lp_ref[...] = e2 - lse

  return pl.pallas_call(
      body,
      grid=(N // BN,),
      in_specs=[
          pl.BlockSpec((BN, K), lambda i: (i, 0)),
          pl.BlockSpec((1, K), lambda i: (0, 0)),
      ],
      out_specs=[
          pl.BlockSpec((BN, K), lambda i: (i, 0)),
          pl.BlockSpec((BN, K), lambda i: (i, 0)),
      ],
      out_shape=[
          jax.ShapeDtypeStruct((N, K), jnp.float32),
          jax.ShapeDtypeStruct((N, K), jnp.float32),
      ],
  )(agg2, b2)


def kernel(x, edge_index, adj_weight, W1, b1, sel_logits, W2, b2, temp):
  N, F = x.shape
  src = edge_index[0]
  dst = edge_index[1]
  selr = (sel_logits / temp).reshape(1, F).astype(jnp.float32)

  support, fs2 = _tc1(x, W1, selr)
  agg = _sc_agg(src, dst, adj_weight, support)
  embed1, support2 = _tc2(agg, b1.reshape(1, -1), W2)
  agg2 = _sc_agg(src, dst, adj_weight, support2)
  embed2, logp = _tc3(agg2, b2.reshape(1, -1))
  return logp, embed1, embed2, fs2.reshape(-1)


# speedup vs baseline: 1.1857x; 1.1857x over previous
import functools

import jax
import jax.numpy as jnp
from jax import lax
from jax.experimental import pallas as pl
from jax.experimental.pallas import tpu as pltpu
from jax.experimental.pallas import tpu_sc as plsc

NT = 16


def _sc_agg_build(E, N, CH):
  EPT = E // NT
  NR = N // NT
  NB = 5 if CH >= 128 else 25
  CKE = NB * 16
  NCH = EPT // CKE
  assert EPT % CKE == 0 and NCH % 2 == 1
  mesh = plsc.VectorSubcoreMesh(core_axis_name="c", subcore_axis_name="s")

  @functools.partial(
      pl.kernel,
      mesh=mesh,
      compiler_params=pltpu.CompilerParams(needs_layout_passes=False,
                                           use_tc_tiling_on_sc=False),
      out_type=jax.ShapeDtypeStruct((2, N, CH), jnp.float32),
      scratch_types=[
          pltpu.VMEM((EPT,), jnp.int32),
          pltpu.VMEM((EPT,), jnp.int32),
          pltpu.VMEM((EPT,), jnp.float32),
          pltpu.VMEM((CKE, CH), jnp.float32),
          pltpu.VMEM((CKE, CH), jnp.float32),
          pltpu.VMEM_SHARED((N, CH), jnp.float32),
          pltpu.SemaphoreType.DMA,
          pltpu.SemaphoreType.DMA,
          pltpu.SemaphoreType.DMA,
          pltpu.SemaphoreType.DMA,
      ],
  )
  def agg_kernel(src_hbm, dst_hbm, w_hbm, sup_hbm, out_hbm,
                 sidx, didx, wbuf, rows0, rows1, acc,
                 gsem0, gsem1, ssem0, ssem1):
    cid = lax.axis_index("c")
    sid = lax.axis_index("s")
    tbl = sup_hbm.at[cid]
    e0 = sid * EPT
    row0 = sid * NR
    z16i = jnp.zeros((16,), jnp.int32)
    z16f = jnp.zeros((16,), jnp.float32)
    iota16 = lax.iota(jnp.int32, 16)

    pltpu.sync_copy(src_hbm.at[pl.ds(e0, EPT)], sidx)
    pltpu.sync_copy(dst_hbm.at[pl.ds(e0, EPT)], didx)
    pltpu.sync_copy(w_hbm.at[pl.ds(e0, EPT)], wbuf)

    def zero(i, carry):
      r = i // (CH // 16)
      g = i % (CH // 16)
      plsc.store_scatter(rows0, [z16i + r, iota16 + g * 16], z16f)
      return carry
    lax.fori_loop(0, (CKE * CH) // 16, zero, 0)
    nfull, rem = divmod(NR, CKE)
    for j in range(nfull):
      pltpu.sync_copy(rows0, acc.at[pl.ds(row0 + j * CKE, CKE)])
    if rem:
      pltpu.sync_copy(rows0.at[pl.ds(0, rem)],
                      acc.at[pl.ds(row0 + nfull * CKE, rem)])
    plsc.subcore_barrier()

    def fire_gathers(k, buf, sem):
      base = k * CKE
      for j in range(NB):
        svec = sidx[pl.ds(base + j * 16, 16)]
        pltpu.async_copy(tbl.at[svec], buf.at[pl.ds(j * 16, 16)], sem)

    def wait_gathers(k, buf, sem):
      base = k * CKE
      for j in range(NB):
        svec = sidx[pl.ds(base + j * 16, 16)]
        pltpu.make_async_copy(tbl.at[svec], buf.at[pl.ds(j * 16, 16)],
                              sem).wait()

    def fire_scatters(k, buf, sem):
      base = k * CKE
      for j in range(NB):
        dvec = didx[pl.ds(base + j * 16, 16)]
        pltpu.async_copy(buf.at[pl.ds(j * 16, 16)], acc.at[dvec], sem,
                         add=True)

    def wait_scatters(k, buf, sem):
      base = k * CKE
      for j in range(NB):
        dvec = didx[pl.ds(base + j * 16, 16)]
        pltpu.make_async_copy(buf.at[pl.ds(j * 16, 16)], acc.at[dvec],
                              sem).wait()

    def compute(k, buf):
      base = k * CKE

      def blk(j, carry):
        w16 = wbuf[pl.ds(base + j * 16, 16)]
        e16 = iota16 + j * 16
        for c in range(CH):
          colspl = z16i + c
          v = plsc.load_gather(buf, [e16, colspl])
          plsc.store_scatter(buf, [e16, colspl], v * w16)
        return carry
      lax.fori_loop(0, NB, blk, 0)

    fire_gathers(0, rows0, gsem0)

    def chunk(k, carry):
      def phase(cur, oth, gsc, gso, ssc, sso):
        wait_gathers(k, cur, gsc)

        @pl.when(k > 0)
        def _():
          wait_scatters(k - 1, oth, sso)

        @pl.when(k < NCH - 1)
        def _():
          fire_gathers(k + 1, oth, gso)
        compute(k, cur)
        fire_scatters(k, cur, ssc)

      @pl.when(k % 2 == 0)
      def _():
        phase(rows0, rows1, gsem0, gsem1, ssem0, ssem1)

      @pl.when(k % 2 == 1)
      def _():
        phase(rows1, rows0, gsem1, gsem0, ssem1, ssem0)
      return carry
    lax.fori_loop(0, NCH, chunk, 0)
    wait_scatters(NCH - 1, rows0, ssem0)

    plsc.subcore_barrier()
    nfull, rem = divmod(NR, CKE)
    for j in range(nfull):
      pltpu.sync_copy(acc.at[pl.ds(row0 + j * CKE, CKE)],
                      out_hbm.at[cid, pl.ds(row0 + j * CKE, CKE)])
    if rem:
      pltpu.sync_copy(acc.at[pl.ds(row0 + nfull * CKE, rem)],
                      out_hbm.at[cid, pl.ds(row0 + nfull * CKE, rem)])

  return agg_kernel


def _sc_agg(src, dst, w, sup):
  E = src.shape[0]
  _, N, CH = sup.shape
  fn = _sc_agg_build(E, N, CH)
  return fn(src, dst, w, sup)


def _tc1(x, W1, selr):
  N, F = x.shape
  H = W1.shape[1]
  BN = N // 10
  HH = H // 2

  def body(x_ref, w_ref, s_ref, sup_ref, fs_ref):
    fs = jax.nn.sigmoid(s_ref[...])
    fs_ref[...] = fs
    res = jnp.dot(x_ref[...] * fs, w_ref[...],
                  preferred_element_type=jnp.float32)
    sup_ref[0] = res[:, :HH]
    sup_ref[1] = res[:, HH:]

  return pl.pallas_call(
      body,
      grid=(N // BN,),
      in_specs=[
          pl.BlockSpec((BN, F), lambda i: (i, 0)),
          pl.BlockSpec((F, H), lambda i: (0, 0)),
          pl.BlockSpec((1, F), lambda i: (0, 0)),
      ],
      out_specs=[
          pl.BlockSpec((2, BN, HH), lambda i: (0, i, 0)),
          pl.BlockSpec((1, F), lambda i: (0, 0)),
      ],
      out_shape=[
          jax.ShapeDtypeStruct((2, N, HH), jnp.float32),
          jax.ShapeDtypeStruct((1, F), jnp.float32),
      ],
  )(x, W1, selr)


def _tc2(agg, b1, W2):
  _, N, HH = agg.shape
  H = 2 * HH
  K = W2.shape[1]
  KH = K // 2
  BN = N // 10

  def body(a_ref, b_ref, w_ref, e1_ref, s2_ref):
    e1 = jnp.concatenate([a_ref[0], a_ref[1]], axis=1) + b_ref[...]
    e1_ref[...] = e1
    res = jnp.dot(jnp.maximum(e1, 0.0), w_ref[...],
                  preferred_element_type=jnp.float32)
    s2_ref[0] = res[:, :KH]
    s2_ref[1] = res[:, KH:]

  return pl.pallas_call(
      body,
      grid=(N // BN,),
      in_specs=[
          pl.BlockSpec((2, BN, HH), lambda i: (0, i, 0)),
          pl.BlockSpec((1, H), lambda i: (0, 0)),
          pl.BlockSpec((H, K), lambda i: (0, 0)),
      ],
      out_specs=[
          pl.BlockSpec((BN, H), lambda i: (i, 0)),
          pl.BlockSpec((2, BN, KH), lambda i: (0, i, 0)),
      ],
      out_shape=[
          jax.ShapeDtypeStruct((N, H), jnp.float32),
          jax.ShapeDtypeStruct((2, N, KH), jnp.float32),
      ],
  )(agg, b1, W2)


def _tc3(agg2, b2):
  _, N, KH = agg2.shape
  K = 2 * KH
  BN = N // 10

  def body(a_ref, b_ref, e2_ref, lp_ref):
    e2 = jnp.concatenate([a_ref[0], a_ref[1]], axis=1) + b_ref[...]
    e2_ref[...] = e2
    m = jnp.max(e2, axis=1, keepdims=True)
    lse = jnp.log(jnp.sum(jnp.exp(e2 - m), axis=1, keepdims=True)) + m
    lp_ref[...] = e2 - lse

  return pl.pallas_call(
      body,
      grid=(N // BN,),
      in_specs=[
          pl.BlockSpec((2, BN, KH), lambda i: (0, i, 0)),
          pl.BlockSpec((1, K), lambda i: (0, 0)),
      ],
      out_specs=[
          pl.BlockSpec((BN, K), lambda i: (i, 0)),
          pl.BlockSpec((BN, K), lambda i: (i, 0)),
      ],
      out_shape=[
          jax.ShapeDtypeStruct((N, K), jnp.float32),
          jax.ShapeDtypeStruct((N, K), jnp.float32),
      ],
  )(agg2, b2)


def kernel(x, edge_index, adj_weight, W1, b1, sel_logits, W2, b2, temp):
  N, F = x.shape
  src = edge_index[0]
  dst = edge_index[1]
  selr = (sel_logits / temp).reshape(1, F).astype(jnp.float32)

  support, fs2 = _tc1(x, W1, selr)
  agg = _sc_agg(src, dst, adj_weight, support)
  embed1, support2 = _tc2(agg, b1.reshape(1, -1), W2)
  agg2 = _sc_agg(src, dst, adj_weight, support2)
  embed2, logp = _tc3(agg2, b2.reshape(1, -1))
  return logp, embed1, embed2, fs2.reshape(-1)
